# initial kernel scaffold (unmeasured)
import jax
import jax.numpy as jnp
from jax import lax
from jax.experimental import pallas as pl
from jax.experimental.pallas import tpu as pltpu


def kernel(ids, E):
    v_local, d = E.shape
    t = ids.shape[0]

    my_x = lax.axis_index("x")
    lids = ids - my_x * v_local
    mask = (lids >= 0) & (lids < v_local)
    rows = jnp.take(E, jnp.clip(lids, 0, v_local - 1), axis=0)
    partial = jnp.where(mask[:, None], rows, 0.0)

    def body(p_ref, out_ref, comm_ref, send_sem, recv_sem):
        x = lax.axis_index("x")
        y = lax.axis_index("y")
        z = lax.axis_index("z")
        peer = (1 - x, y, z)

        barrier = pltpu.get_barrier_semaphore()
        pl.semaphore_signal(
            barrier, inc=1, device_id=peer,
            device_id_type=pl.DeviceIdType.MESH,
        )
        pl.semaphore_wait(barrier, 1)

        rdma = pltpu.make_async_remote_copy(
            src_ref=p_ref,
            dst_ref=comm_ref,
            send_sem=send_sem,
            recv_sem=recv_sem,
            device_id=peer,
            device_id_type=pl.DeviceIdType.MESH,
        )
        rdma.start()
        rdma.wait()

        out_ref[...] = p_ref[...] + comm_ref[...]

    return pl.pallas_call(
        body,
        out_shape=jax.ShapeDtypeStruct((t, d), jnp.float32),
        in_specs=[pl.BlockSpec(memory_space=pltpu.VMEM)],
        out_specs=pl.BlockSpec(memory_space=pltpu.VMEM),
        scratch_shapes=[
            pltpu.VMEM((t, d), jnp.float32),
            pltpu.SemaphoreType.DMA,
            pltpu.SemaphoreType.DMA,
        ],
        compiler_params=pltpu.CompilerParams(collective_id=0),
    )(partial)


# baseline (device time: 297495 ns/iter reference)
import jax
import jax.numpy as jnp
from jax import lax
from jax.experimental import pallas as pl
from jax.experimental.pallas import tpu as pltpu

_K = 8


def kernel(ids, E):
    v_local, d = E.shape
    t = ids.shape[0]

    my_x = lax.axis_index("x")
    lids = ids - my_x * v_local
    cl = jnp.clip(lids, 0, v_local - 1).astype(jnp.int32)
    maskf = ((lids >= 0) & (lids < v_local)).astype(jnp.float32)[:, None]

    def body(cl_ref, mask_ref, e_ref, out_ref, part_ref, comm_ref,
             gsems, send_sem, recv_sem):
        x = lax.axis_index("x")
        y = lax.axis_index("y")
        z = lax.axis_index("z")
        peer = (1 - x, y, z)

        def row_copy(i):
            return pltpu.make_async_copy(
                e_ref.at[pl.ds(cl_ref[i], 1), :],
                part_ref.at[pl.ds(i, 1), :],
                gsems.at[lax.rem(i, _K)],
            )

        def gather_step(i, carry):
            @pl.when(i >= _K)
            def _():
                row_copy(i - _K).wait()
            row_copy(i).start()
            return carry

        lax.fori_loop(0, t, gather_step, 0)

        def drain_step(j, carry):
            row_copy(t - _K + j).wait()
            return carry

        lax.fori_loop(0, _K, drain_step, 0)

        part_ref[...] = part_ref[...] * mask_ref[...]

        barrier = pltpu.get_barrier_semaphore()
        pl.semaphore_signal(
            barrier, inc=1, device_id=peer,
            device_id_type=pl.DeviceIdType.MESH,
        )
        pl.semaphore_wait(barrier, 1)

        rdma = pltpu.make_async_remote_copy(
            src_ref=part_ref,
            dst_ref=comm_ref,
            send_sem=send_sem,
            recv_sem=recv_sem,
            device_id=peer,
            device_id_type=pl.DeviceIdType.MESH,
        )
        rdma.start()
        rdma.wait()

        out_ref[...] = part_ref[...] + comm_ref[...]

    return pl.pallas_call(
        body,
        out_shape=jax.ShapeDtypeStruct((t, d), jnp.float32),
        in_specs=[
            pl.BlockSpec(memory_space=pltpu.SMEM),
            pl.BlockSpec(memory_space=pltpu.VMEM),
            pl.BlockSpec(memory_space=pl.ANY),
        ],
        out_specs=pl.BlockSpec(memory_space=pltpu.VMEM),
        scratch_shapes=[
            pltpu.VMEM((t, d), jnp.float32),
            pltpu.VMEM((t, d), jnp.float32),
            pltpu.SemaphoreType.DMA((_K,)),
            pltpu.SemaphoreType.DMA,
            pltpu.SemaphoreType.DMA,
        ],
        compiler_params=pltpu.CompilerParams(collective_id=0),
    )(cl, maskf, E)


# device time: 99081 ns/iter; 3.0025x vs baseline; 3.0025x over previous
import jax
import jax.numpy as jnp
from jax import lax
from jax.experimental import pallas as pl
from jax.experimental.pallas import tpu as pltpu

_K = 8
_NR = 8
_CW = 4
_CCW = 3


def _ring_coords(p):
    y = (p >= _NR // 2).astype(jnp.int32)
    z = jnp.where(p < _NR // 2, p, _NR - 1 - p)
    return y, z


def kernel(ids, E):
    v_local, d = E.shape
    t = ids.shape[0]
    c = t // _NR

    my_x = lax.axis_index("x")
    my_y = lax.axis_index("y")
    my_z = lax.axis_index("z")
    r = jnp.where(my_y == 0, my_z, _NR - 1 - my_z).astype(jnp.int32)

    lids = ids - my_x * v_local
    cl_all = jnp.clip(lids, 0, v_local - 1).astype(jnp.int32)
    mask_all = ((lids >= 0) & (lids < v_local)).astype(jnp.float32)[:, None]
    cl = lax.dynamic_slice(cl_all, (r * c,), (c,))
    maskf = lax.dynamic_slice(mask_all, (r * c, 0), (c, 1))

    def body(cl_ref, mask_ref, e_ref, out_ref, part_ref, comm_ref,
             gsems, xs_sem, xr_sem, cw_send, cw_recv, ccw_send, ccw_recv):
        x = lax.axis_index("x")
        y = lax.axis_index("y")
        z = lax.axis_index("z")
        rr = jnp.where(y == 0, z, _NR - 1 - z).astype(jnp.int32)
        xpeer = (1 - x, y, z)
        ry, rz = _ring_coords(lax.rem(rr + 1, _NR))
        ly, lz = _ring_coords(lax.rem(rr + _NR - 1, _NR))
        right = (x, ry, rz)
        left = (x, ly, lz)

        def row_copy(i):
            return pltpu.make_async_copy(
                e_ref.at[pl.ds(cl_ref[i], 1), :],
                part_ref.at[pl.ds(i, 1), :],
                gsems.at[lax.rem(i, _K)],
            )

        def gather_step(i, carry):
            @pl.when(i >= _K)
            def _():
                row_copy(i - _K).wait()
            row_copy(i).start()
            return carry

        lax.fori_loop(0, c, gather_step, 0)

        def drain_step(j, carry):
            row_copy(c - _K + j).wait()
            return carry

        lax.fori_loop(0, _K, drain_step, 0)

        part_ref[...] = part_ref[...] * mask_ref[...]

        barrier = pltpu.get_barrier_semaphore()
        for nbr in (xpeer, left, right):
            pl.semaphore_signal(
                barrier, inc=1, device_id=nbr,
                device_id_type=pl.DeviceIdType.MESH,
            )
        pl.semaphore_wait(barrier, 3)

        xchg = pltpu.make_async_remote_copy(
            src_ref=part_ref,
            dst_ref=comm_ref,
            send_sem=xs_sem,
            recv_sem=xr_sem,
            device_id=xpeer,
            device_id_type=pl.DeviceIdType.MESH,
        )
        xchg.start()
        xchg.wait()
        out_ref[pl.ds(rr * c, c), :] = part_ref[...] + comm_ref[...]

        def chunk(idx):
            return out_ref.at[pl.ds(idx * c, c), :]

        def cw_send_d(h):
            s = lax.rem(rr - h + _NR, _NR)
            return pltpu.make_async_remote_copy(
                src_ref=chunk(s), dst_ref=chunk(s),
                send_sem=cw_send.at[h], recv_sem=cw_recv.at[h],
                device_id=right, device_id_type=pl.DeviceIdType.MESH,
            )

        def cw_recv_d(h):
            s = lax.rem(rr - 1 - h + _NR, _NR)
            return pltpu.make_async_remote_copy(
                src_ref=chunk(s), dst_ref=chunk(s),
                send_sem=cw_send.at[h], recv_sem=cw_recv.at[h],
                device_id=left, device_id_type=pl.DeviceIdType.MESH,
            )

        def ccw_send_d(h):
            s = lax.rem(rr + h, _NR)
            return pltpu.make_async_remote_copy(
                src_ref=chunk(s), dst_ref=chunk(s),
                send_sem=ccw_send.at[h], recv_sem=ccw_recv.at[h],
                device_id=left, device_id_type=pl.DeviceIdType.MESH,
            )

        def ccw_recv_d(h):
            s = lax.rem(rr + 1 + h, _NR)
            return pltpu.make_async_remote_copy(
                src_ref=chunk(s), dst_ref=chunk(s),
                send_sem=ccw_send.at[h], recv_sem=ccw_recv.at[h],
                device_id=right, device_id_type=pl.DeviceIdType.MESH,
            )

        for h in range(_CW):
            cw_send_d(h).start()
            if h < _CCW:
                ccw_send_d(h).start()
            cw_recv_d(h).wait_recv()
            if h < _CCW:
                ccw_recv_d(h).wait_recv()

        for h in range(_CW):
            cw_send_d(h).wait_send()
        for h in range(_CCW):
            ccw_send_d(h).wait_send()

    return pl.pallas_call(
        body,
        out_shape=jax.ShapeDtypeStruct((t, d), jnp.float32),
        in_specs=[
            pl.BlockSpec(memory_space=pltpu.SMEM),
            pl.BlockSpec(memory_space=pltpu.VMEM),
            pl.BlockSpec(memory_space=pl.ANY),
        ],
        out_specs=pl.BlockSpec(memory_space=pltpu.VMEM),
        scratch_shapes=[
            pltpu.VMEM((c, d), jnp.float32),
            pltpu.VMEM((c, d), jnp.float32),
            pltpu.SemaphoreType.DMA((_K,)),
            pltpu.SemaphoreType.DMA,
            pltpu.SemaphoreType.DMA,
            pltpu.SemaphoreType.DMA((_CW,)),
            pltpu.SemaphoreType.DMA((_CW,)),
            pltpu.SemaphoreType.DMA((_CCW,)),
            pltpu.SemaphoreType.DMA((_CCW,)),
        ],
        compiler_params=pltpu.CompilerParams(collective_id=0),
    )(cl, maskf, E)


# device time: 81991 ns/iter; 3.6284x vs baseline; 1.2084x over previous
import jax
import jax.numpy as jnp
from jax import lax
from jax.experimental import pallas as pl
from jax.experimental.pallas import tpu as pltpu

_K = 16
_NR = 8
_CW = 4
_CCW = 4


def _ring_coords(p):
    y = (p >= _NR // 2).astype(jnp.int32)
    z = jnp.where(p < _NR // 2, p, _NR - 1 - p)
    return y, z


def kernel(ids, E):
    v_local, d = E.shape
    t = ids.shape[0]
    c = t // _NR

    my_x = lax.axis_index("x")
    my_y = lax.axis_index("y")
    my_z = lax.axis_index("z")
    r = jnp.where(my_y == 0, my_z, _NR - 1 - my_z).astype(jnp.int32)

    lids = ids - my_x * v_local
    cl_all = jnp.clip(lids, 0, v_local - 1).astype(jnp.int32)
    mask_all = ((lids >= 0) & (lids < v_local)).astype(jnp.float32)[:, None]
    cl = lax.dynamic_slice(cl_all, (r * c,), (c,))
    maskf = lax.dynamic_slice(mask_all, (r * c, 0), (c, 1))

    def body(cl_ref, mask_ref, e_ref, out_ref, part_ref, comm_ref,
             gsems, xs_sem, xr_sem, cw_send, cw_recv, ccw_send, ccw_recv):
        x = lax.axis_index("x")
        y = lax.axis_index("y")
        z = lax.axis_index("z")
        rr = jnp.where(y == 0, z, _NR - 1 - z).astype(jnp.int32)
        xpeer = (1 - x, y, z)
        ry, rz = _ring_coords(lax.rem(rr + 1, _NR))
        ly, lz = _ring_coords(lax.rem(rr + _NR - 1, _NR))
        right = (x, ry, rz)
        left = (x, ly, lz)

        def row_copy(i):
            return pltpu.make_async_copy(
                e_ref.at[pl.ds(cl_ref[i], 1), :],
                part_ref.at[pl.ds(i, 1), :],
                gsems.at[lax.rem(i, _K)],
            )

        def gather_step(i, carry):
            @pl.when(i >= _K)
            def _():
                row_copy(i - _K).wait()
            row_copy(i).start()
            return carry

        lax.fori_loop(0, c, gather_step, 0)

        def drain_step(j, carry):
            row_copy(c - _K + j).wait()
            return carry

        lax.fori_loop(0, _K, drain_step, 0)

        part_ref[...] = part_ref[...] * mask_ref[...]

        barrier = pltpu.get_barrier_semaphore()
        for nbr in (xpeer, left, right):
            pl.semaphore_signal(
                barrier, inc=1, device_id=nbr,
                device_id_type=pl.DeviceIdType.MESH,
            )
        pl.semaphore_wait(barrier, 3)

        xchg = pltpu.make_async_remote_copy(
            src_ref=part_ref,
            dst_ref=comm_ref,
            send_sem=xs_sem,
            recv_sem=xr_sem,
            device_id=xpeer,
            device_id_type=pl.DeviceIdType.MESH,
        )
        xchg.start()
        xchg.wait()
        out_ref[pl.ds(rr * c, c), :] = part_ref[...] + comm_ref[...]

        h2 = c // 2

        def chunk(idx, half):
            if half is None:
                return out_ref.at[pl.ds(idx * c, c), :]
            return out_ref.at[pl.ds(idx * c + half * h2, h2), :]

        def cw_half(h):
            return None if h < _CW - 1 else 0

        def ccw_half(h):
            return None if h < _CCW - 1 else 1

        def cw_send_d(h):
            s = lax.rem(rr - h + _NR, _NR)
            ref = chunk(s, cw_half(h))
            return pltpu.make_async_remote_copy(
                src_ref=ref, dst_ref=ref,
                send_sem=cw_send.at[h], recv_sem=cw_recv.at[h],
                device_id=right, device_id_type=pl.DeviceIdType.MESH,
            )

        def cw_recv_d(h):
            s = lax.rem(rr - 1 - h + _NR, _NR)
            ref = chunk(s, cw_half(h))
            return pltpu.make_async_remote_copy(
                src_ref=ref, dst_ref=ref,
                send_sem=cw_send.at[h], recv_sem=cw_recv.at[h],
                device_id=left, device_id_type=pl.DeviceIdType.MESH,
            )

        def ccw_send_d(h):
            s = lax.rem(rr + h, _NR)
            ref = chunk(s, ccw_half(h))
            return pltpu.make_async_remote_copy(
                src_ref=ref, dst_ref=ref,
                send_sem=ccw_send.at[h], recv_sem=ccw_recv.at[h],
                device_id=left, device_id_type=pl.DeviceIdType.MESH,
            )

        def ccw_recv_d(h):
            s = lax.rem(rr + 1 + h, _NR)
            ref = chunk(s, ccw_half(h))
            return pltpu.make_async_remote_copy(
                src_ref=ref, dst_ref=ref,
                send_sem=ccw_send.at[h], recv_sem=ccw_recv.at[h],
                device_id=right, device_id_type=pl.DeviceIdType.MESH,
            )

        for h in range(_CW):
            cw_send_d(h).start()
            ccw_send_d(h).start()
            cw_recv_d(h).wait_recv()
            ccw_recv_d(h).wait_recv()

        for h in range(_CW):
            cw_send_d(h).wait_send()
        for h in range(_CCW):
            ccw_send_d(h).wait_send()

    return pl.pallas_call(
        body,
        out_shape=jax.ShapeDtypeStruct((t, d), jnp.float32),
        in_specs=[
            pl.BlockSpec(memory_space=pltpu.SMEM),
            pl.BlockSpec(memory_space=pltpu.VMEM),
            pl.BlockSpec(memory_space=pl.ANY),
        ],
        out_specs=pl.BlockSpec(memory_space=pltpu.VMEM),
        scratch_shapes=[
            pltpu.VMEM((c, d), jnp.float32),
            pltpu.VMEM((c, d), jnp.float32),
            pltpu.SemaphoreType.DMA((_K,)),
            pltpu.SemaphoreType.DMA,
            pltpu.SemaphoreType.DMA,
            pltpu.SemaphoreType.DMA((_CW,)),
            pltpu.SemaphoreType.DMA((_CW,)),
            pltpu.SemaphoreType.DMA((_CCW,)),
            pltpu.SemaphoreType.DMA((_CCW,)),
        ],
        compiler_params=pltpu.CompilerParams(collective_id=0),
    )(cl, maskf, E)


# device time: 67852 ns/iter; 4.3845x vs baseline; 1.2084x over previous
import jax
import jax.numpy as jnp
from jax import lax
from jax.experimental import pallas as pl
from jax.experimental.pallas import tpu as pltpu

_K = 16
_NR = 8
_SUB = 2
_NSLOT = 7 * _SUB // 2

_G = [s for pair in zip(range(_SUB // 2), range(_SUB // 2, _SUB)) for s in pair]


def _ring_coords(p):
    y = (p >= _NR // 2).astype(jnp.int32)
    z = jnp.where(p < _NR // 2, p, _NR - 1 - p)
    return y, z


def kernel(ids, E):
    v_local, d = E.shape
    t = ids.shape[0]
    c = t // _NR
    sz = c // _SUB

    my_x = lax.axis_index("x")
    my_y = lax.axis_index("y")
    my_z = lax.axis_index("z")
    r = jnp.where(my_y == 0, my_z, _NR - 1 - my_z).astype(jnp.int32)

    lids = ids - my_x * v_local
    cl_all = jnp.clip(lids, 0, v_local - 1).astype(jnp.int32)
    mask_all = ((lids >= 0) & (lids < v_local)).astype(jnp.float32)[:, None]
    cl = lax.dynamic_slice(cl_all, (r * c,), (c,))
    maskf = lax.dynamic_slice(mask_all, (r * c, 0), (c, 1))

    def body(cl_ref, mask_ref, e_ref, out_ref, part_ref, comm_ref,
             gsems, xs_sems, xr_sems, cw_s, cw_r, ccw_s, ccw_r):
        x = lax.axis_index("x")
        y = lax.axis_index("y")
        z = lax.axis_index("z")
        rr = jnp.where(y == 0, z, _NR - 1 - z).astype(jnp.int32)
        xpeer = (1 - x, y, z)
        ry, rz = _ring_coords(lax.rem(rr + 1, _NR))
        ly, lz = _ring_coords(lax.rem(rr + _NR - 1, _NR))
        right = (x, ry, rz)
        left = (x, ly, lz)

        barrier = pltpu.get_barrier_semaphore()
        for nbr in (xpeer, left, right):
            pl.semaphore_signal(
                barrier, inc=1, device_id=nbr,
                device_id_type=pl.DeviceIdType.MESH,
            )
        pl.semaphore_wait(barrier, 3)

        def exch_d(s):
            return pltpu.make_async_remote_copy(
                src_ref=part_ref.at[pl.ds(s * sz, sz), :],
                dst_ref=comm_ref.at[pl.ds(s * sz, sz), :],
                send_sem=xs_sems.at[s], recv_sem=xr_sems.at[s],
                device_id=xpeer, device_id_type=pl.DeviceIdType.MESH,
            )

        def sub_slice(chunk_idx, sub):
            return out_ref.at[pl.ds(chunk_idx * c + sub * sz, sz), :]

        def cw_sub(m):
            return m % _SUB

        def ccw_sub(m):
            return (_SUB // 2 + m) % _SUB

        def cw_send_d(m):
            ref = sub_slice(lax.rem(rr - m // _SUB + _NR, _NR), cw_sub(m))
            return pltpu.make_async_remote_copy(
                src_ref=ref, dst_ref=ref,
                send_sem=cw_s.at[m], recv_sem=cw_r.at[m],
                device_id=right, device_id_type=pl.DeviceIdType.MESH,
            )

        def cw_recv_d(m):
            ref = sub_slice(lax.rem(rr - 1 - m // _SUB + _NR, _NR), cw_sub(m))
            return pltpu.make_async_remote_copy(
                src_ref=ref, dst_ref=ref,
                send_sem=cw_s.at[m], recv_sem=cw_r.at[m],
                device_id=left, device_id_type=pl.DeviceIdType.MESH,
            )

        def ccw_send_d(m):
            ref = sub_slice(lax.rem(rr + m // _SUB, _NR), ccw_sub(m))
            return pltpu.make_async_remote_copy(
                src_ref=ref, dst_ref=ref,
                send_sem=ccw_s.at[m], recv_sem=ccw_r.at[m],
                device_id=left, device_id_type=pl.DeviceIdType.MESH,
            )

        def ccw_recv_d(m):
            ref = sub_slice(lax.rem(rr + 1 + m // _SUB, _NR), ccw_sub(m))
            return pltpu.make_async_remote_copy(
                src_ref=ref, dst_ref=ref,
                send_sem=ccw_s.at[m], recv_sem=ccw_r.at[m],
                device_id=right, device_id_type=pl.DeviceIdType.MESH,
            )

        def gather_sub(s):
            def row_copy(i):
                return pltpu.make_async_copy(
                    e_ref.at[pl.ds(cl_ref[i], 1), :],
                    part_ref.at[pl.ds(i, 1), :],
                    gsems.at[lax.rem(i, _K)],
                )

            lo = s * sz

            def gather_step(i, carry):
                @pl.when(i >= lo + _K)
                def _():
                    row_copy(i - _K).wait()
                row_copy(i).start()
                return carry

            lax.fori_loop(lo, lo + sz, gather_step, 0)

            def drain_step(j, carry):
                row_copy(lo + sz - _K + j).wait()
                return carry

            lax.fori_loop(0, _K, drain_step, 0)
            part_ref[pl.ds(lo, sz), :] = (
                part_ref[pl.ds(lo, sz), :] * mask_ref[pl.ds(lo, sz), :]
            )

        def process_sub(s):
            exch_d(s).wait_recv()
            out_ref[pl.ds(rr * c + s * sz, sz), :] = (
                part_ref[pl.ds(s * sz, sz), :]
                + comm_ref[pl.ds(s * sz, sz), :]
            )
            cw_send_d(s).start()
            ccw_send_d((s - _SUB // 2) % _SUB).start()

        for i, s in enumerate(_G):
            gather_sub(s)
            exch_d(s).start()
            if i > 0:
                process_sub(_G[i - 1])
        process_sub(_G[-1])

        for m in range(_SUB, _NSLOT):
            cw_recv_d(m - _SUB).wait_recv()
            cw_send_d(m).start()
            ccw_recv_d(m - _SUB).wait_recv()
            ccw_send_d(m).start()

        for m in range(_NSLOT - _SUB, _NSLOT):
            cw_recv_d(m).wait_recv()
            ccw_recv_d(m).wait_recv()

        for s in range(_SUB):
            exch_d(s).wait_send()
        for m in range(_NSLOT):
            cw_send_d(m).wait_send()
            ccw_send_d(m).wait_send()

    return pl.pallas_call(
        body,
        out_shape=jax.ShapeDtypeStruct((t, d), jnp.float32),
        in_specs=[
            pl.BlockSpec(memory_space=pltpu.SMEM),
            pl.BlockSpec(memory_space=pltpu.VMEM),
            pl.BlockSpec(memory_space=pl.ANY),
        ],
        out_specs=pl.BlockSpec(memory_space=pltpu.VMEM),
        scratch_shapes=[
            pltpu.VMEM((c, d), jnp.float32),
            pltpu.VMEM((c, d), jnp.float32),
            pltpu.SemaphoreType.DMA((_K,)),
            pltpu.SemaphoreType.DMA((_SUB,)),
            pltpu.SemaphoreType.DMA((_SUB,)),
            pltpu.SemaphoreType.DMA((_NSLOT,)),
            pltpu.SemaphoreType.DMA((_NSLOT,)),
            pltpu.SemaphoreType.DMA((_NSLOT,)),
            pltpu.SemaphoreType.DMA((_NSLOT,)),
        ],
        compiler_params=pltpu.CompilerParams(collective_id=0),
    )(cl, maskf, E)


# device time: 61816 ns/iter; 4.8126x vs baseline; 1.0976x over previous
import jax
import jax.numpy as jnp
from jax import lax
from jax.experimental import pallas as pl
from jax.experimental.pallas import tpu as pltpu

_K = 16
_NR = 8
_SUB = 4
_NSLOT = 7 * _SUB // 2

_G = [s for pair in zip(range(_SUB // 2), range(_SUB // 2, _SUB)) for s in pair]


def _ring_coords(p):
    y = (p >= _NR // 2).astype(jnp.int32)
    z = jnp.where(p < _NR // 2, p, _NR - 1 - p)
    return y, z


def kernel(ids, E):
    v_local, d = E.shape
    t = ids.shape[0]
    c = t // _NR
    sz = c // _SUB

    my_x = lax.axis_index("x")
    my_y = lax.axis_index("y")
    my_z = lax.axis_index("z")
    r = jnp.where(my_y == 0, my_z, _NR - 1 - my_z).astype(jnp.int32)

    lids = ids - my_x * v_local
    cl_all = jnp.clip(lids, 0, v_local - 1).astype(jnp.int32)
    mask_all = ((lids >= 0) & (lids < v_local)).astype(jnp.float32)[:, None]
    cl = lax.dynamic_slice(cl_all, (r * c,), (c,))
    maskf = lax.dynamic_slice(mask_all, (r * c, 0), (c, 1))

    def body(cl_ref, mask_ref, e_ref, out_ref, part_ref, comm_ref,
             gsems, xs_sems, xr_sems, cw_s, cw_r, ccw_s, ccw_r):
        x = lax.axis_index("x")
        y = lax.axis_index("y")
        z = lax.axis_index("z")
        rr = jnp.where(y == 0, z, _NR - 1 - z).astype(jnp.int32)
        xpeer = (1 - x, y, z)
        ry, rz = _ring_coords(lax.rem(rr + 1, _NR))
        ly, lz = _ring_coords(lax.rem(rr + _NR - 1, _NR))
        right = (x, ry, rz)
        left = (x, ly, lz)

        barrier = pltpu.get_barrier_semaphore()
        for nbr in (xpeer, left, right):
            pl.semaphore_signal(
                barrier, inc=1, device_id=nbr,
                device_id_type=pl.DeviceIdType.MESH,
            )
        pl.semaphore_wait(barrier, 3)

        def exch_d(s):
            return pltpu.make_async_remote_copy(
                src_ref=part_ref.at[pl.ds(s * sz, sz), :],
                dst_ref=comm_ref.at[pl.ds(s * sz, sz), :],
                send_sem=xs_sems.at[s], recv_sem=xr_sems.at[s],
                device_id=xpeer, device_id_type=pl.DeviceIdType.MESH,
            )

        def sub_slice(chunk_idx, sub):
            return out_ref.at[pl.ds(chunk_idx * c + sub * sz, sz), :]

        def cw_sub(m):
            return m % _SUB

        def ccw_sub(m):
            return (_SUB // 2 + m) % _SUB

        def cw_send_d(m):
            ref = sub_slice(lax.rem(rr - m // _SUB + _NR, _NR), cw_sub(m))
            return pltpu.make_async_remote_copy(
                src_ref=ref, dst_ref=ref,
                send_sem=cw_s.at[m], recv_sem=cw_r.at[m],
                device_id=right, device_id_type=pl.DeviceIdType.MESH,
            )

        def cw_recv_d(m):
            ref = sub_slice(lax.rem(rr - 1 - m // _SUB + _NR, _NR), cw_sub(m))
            return pltpu.make_async_remote_copy(
                src_ref=ref, dst_ref=ref,
                send_sem=cw_s.at[m], recv_sem=cw_r.at[m],
                device_id=left, device_id_type=pl.DeviceIdType.MESH,
            )

        def ccw_send_d(m):
            ref = sub_slice(lax.rem(rr + m // _SUB, _NR), ccw_sub(m))
            return pltpu.make_async_remote_copy(
                src_ref=ref, dst_ref=ref,
                send_sem=ccw_s.at[m], recv_sem=ccw_r.at[m],
                device_id=left, device_id_type=pl.DeviceIdType.MESH,
            )

        def ccw_recv_d(m):
            ref = sub_slice(lax.rem(rr + 1 + m // _SUB, _NR), ccw_sub(m))
            return pltpu.make_async_remote_copy(
                src_ref=ref, dst_ref=ref,
                send_sem=ccw_s.at[m], recv_sem=ccw_r.at[m],
                device_id=right, device_id_type=pl.DeviceIdType.MESH,
            )

        def gather_sub(s):
            def row_copy(i):
                return pltpu.make_async_copy(
                    e_ref.at[pl.ds(cl_ref[i], 1), :],
                    part_ref.at[pl.ds(i, 1), :],
                    gsems.at[lax.rem(i, _K)],
                )

            lo = s * sz

            def gather_step(i, carry):
                @pl.when(i >= lo + _K)
                def _():
                    row_copy(i - _K).wait()
                row_copy(i).start()
                return carry

            lax.fori_loop(lo, lo + sz, gather_step, 0)

            def drain_step(j, carry):
                row_copy(lo + sz - _K + j).wait()
                return carry

            lax.fori_loop(0, _K, drain_step, 0)
            part_ref[pl.ds(lo, sz), :] = (
                part_ref[pl.ds(lo, sz), :] * mask_ref[pl.ds(lo, sz), :]
            )

        def process_sub(s):
            exch_d(s).wait_recv()
            out_ref[pl.ds(rr * c + s * sz, sz), :] = (
                part_ref[pl.ds(s * sz, sz), :]
                + comm_ref[pl.ds(s * sz, sz), :]
            )
            cw_send_d(s).start()
            ccw_send_d((s - _SUB // 2) % _SUB).start()

        for i, s in enumerate(_G):
            gather_sub(s)
            exch_d(s).start()
            if i > 0:
                process_sub(_G[i - 1])
        process_sub(_G[-1])

        for m in range(_SUB, _NSLOT):
            cw_recv_d(m - _SUB).wait_recv()
            cw_send_d(m).start()
            ccw_recv_d(m - _SUB).wait_recv()
            ccw_send_d(m).start()

        for m in range(_NSLOT - _SUB, _NSLOT):
            cw_recv_d(m).wait_recv()
            ccw_recv_d(m).wait_recv()

        for s in range(_SUB):
            exch_d(s).wait_send()
        for m in range(_NSLOT):
            cw_send_d(m).wait_send()
            ccw_send_d(m).wait_send()

    return pl.pallas_call(
        body,
        out_shape=jax.ShapeDtypeStruct((t, d), jnp.float32),
        in_specs=[
            pl.BlockSpec(memory_space=pltpu.SMEM),
            pl.BlockSpec(memory_space=pltpu.VMEM),
            pl.BlockSpec(memory_space=pl.ANY),
        ],
        out_specs=pl.BlockSpec(memory_space=pltpu.VMEM),
        scratch_shapes=[
            pltpu.VMEM((c, d), jnp.float32),
            pltpu.VMEM((c, d), jnp.float32),
            pltpu.SemaphoreType.DMA((_K,)),
            pltpu.SemaphoreType.DMA((_SUB,)),
            pltpu.SemaphoreType.DMA((_SUB,)),
            pltpu.SemaphoreType.DMA((_NSLOT,)),
            pltpu.SemaphoreType.DMA((_NSLOT,)),
            pltpu.SemaphoreType.DMA((_NSLOT,)),
            pltpu.SemaphoreType.DMA((_NSLOT,)),
        ],
        compiler_params=pltpu.CompilerParams(collective_id=0),
    )(cl, maskf, E)
